# baseline (speedup 1.0000x reference)
import jax
import jax.numpy as jnp
from jax.experimental import pallas as pl


def _body(Wh_ref, Wwr_ref, out_ref):
    i = pl.program_id(0)
    j = pl.program_id(1)
    B, _, _, W, D = out_ref.shape
    H = pl.num_programs(0)
    wh_row = Wh_ref[pl.ds(H - 1 + i - j, 1), :]
    ew = Wwr_ref[pl.ds(W - 1 - j, W), :]
    val = wh_row + ew
    out_ref[...] = jnp.broadcast_to(val[None, None, None], (B, 1, 1, W, D))


def kernel(x, Wh, Ww):
    B, C, H, W = x.shape
    D = Wh.shape[1]
    Wwr = Ww[::-1]
    return pl.pallas_call(
        _body,
        grid=(H, H),
        in_specs=[
            pl.BlockSpec((2 * H - 1, D), lambda i, j: (0, 0)),
            pl.BlockSpec((2 * W - 1, D), lambda i, j: (0, 0)),
        ],
        out_specs=pl.BlockSpec((B, 1, 1, W, D), lambda i, j: (0, i, j, 0, 0)),
        out_shape=jax.ShapeDtypeStruct((B, H, H, W, D), jnp.float32),
    )(Wh, Wwr)


# speedup vs baseline: 7.7585x; 7.7585x over previous
import jax
import jax.numpy as jnp
from jax.experimental import pallas as pl
from jax.experimental.pallas import tpu as pltpu


def _body(Whr_ref, Wwr_ref, out_ref, eh_ref, ew_ref):
    b = pl.program_id(0)
    ib = pl.program_id(1)
    _, BI, H, W, D = out_ref.shape

    @pl.when(jnp.logical_and(b == 0, ib == 0))
    def _init():
        for i in range(H):
            eh_ref[i] = Whr_ref[pl.ds(H - 1 - i, H), :]
        for j in range(W):
            ew_ref[j] = Wwr_ref[pl.ds(W - 1 - j, W), :]

    eh = eh_ref[pl.ds(ib * BI, BI)]
    ew = ew_ref[...]
    out_ref[0] = eh[:, :, None, :] + ew[None, :, :, :]


def kernel(x, Wh, Ww):
    B, C, H, W = x.shape
    D = Wh.shape[1]
    BI = 8
    Whr = Wh[::-1]
    Wwr = Ww[::-1]
    return pl.pallas_call(
        _body,
        grid=(B, H // BI),
        in_specs=[
            pl.BlockSpec((2 * H - 1, D), lambda b, ib: (0, 0)),
            pl.BlockSpec((2 * W - 1, D), lambda b, ib: (0, 0)),
        ],
        out_specs=pl.BlockSpec((1, BI, H, W, D), lambda b, ib: (b, ib, 0, 0, 0)),
        out_shape=jax.ShapeDtypeStruct((B, H, H, W, D), jnp.float32),
        scratch_shapes=[
            pltpu.VMEM((H, H, D), jnp.float32),
            pltpu.VMEM((W, W, D), jnp.float32),
        ],
    )(Whr, Wwr)
